# Initial kernel scaffold; baseline (speedup 1.0000x reference)
import functools

import jax
import jax.numpy as jnp
from jax import lax
from jax.experimental import pallas as pl
from jax.experimental.pallas import tpu as pltpu
from jax.experimental.pallas import tpu_sc as plsc

S = 4096
D = 1024
E = 8
DFF = 4096
CAP = 640
RPE = 656
NROWS = E * RPE
SENT = 648
TB = 512
LW = 128
NJ = 4
DFB = DFF // NJ

NW = 32
CH = 64
NCH = (S // NW) // CH

ROUTER_WEIGHT = 0.001
EXPERTS_WEIGHT = 0.01


def _router_body(h_ref, gw_ref, dst_ref, loss_ref, acc_ref):
    i = pl.program_id(0)
    nb = pl.num_programs(0)

    @pl.when(i == 0)
    def _():
        acc_ref[...] = jnp.zeros_like(acc_ref)

    h = h_ref[...]
    logits = jnp.dot(h, gw_ref[...], preferred_element_type=jnp.float32)
    lane = lax.broadcasted_iota(jnp.int32, (TB, LW), 1)
    neg_inf = jnp.float32(-jnp.inf)
    logits = jnp.where(lane < E, logits, neg_inf)

    m1 = jnp.max(logits, axis=1, keepdims=True)
    i1 = jnp.min(jnp.where(logits == m1, lane, LW), axis=1, keepdims=True)
    l2 = jnp.where(lane == i1, neg_inf, logits)
    m2 = jnp.max(l2, axis=1, keepdims=True)
    i2 = jnp.min(jnp.where(l2 == m2, lane, LW), axis=1, keepdims=True)

    ex = jnp.exp(logits - m1)
    se = jnp.sum(ex, axis=1, keepdims=True)
    probs = ex / se
    lse = m1 + jnp.log(se)

    oh1 = (lane == i1).astype(jnp.float32)
    oh2 = (lane == i2).astype(jnp.float32)
    d_oh = oh1 + oh2

    r_i = lax.broadcasted_iota(jnp.int32, (TB, TB), 0)
    c_i = lax.broadcasted_iota(jnp.int32, (TB, TB), 1)
    tri = (r_i >= c_i).astype(jnp.float32)

    pos = jnp.dot(tri, d_oh, preferred_element_type=jnp.float32) + acc_ref[0:1, :]
    pos1 = jnp.sum(pos * oh1, axis=1, keepdims=True)
    pos2 = jnp.sum(pos * oh2, axis=1, keepdims=True)
    cand1 = jnp.where(pos1 <= CAP, i1, -1)
    cand2 = jnp.where(pos2 <= CAP, i2, -1)
    assigned = jnp.maximum(cand1, cand2)

    a_oh = (lane == assigned).astype(jnp.float32)
    spos = jnp.dot(tri, a_oh, preferred_element_type=jnp.float32) + acc_ref[1:2, :]
    slot = jnp.sum(spos * a_oh, axis=1, keepdims=True).astype(jnp.int32) - 1
    dst_ref[...] = jnp.where(assigned >= 0, assigned * RPE + slot, SENT)

    acc_ref[0:1, :] = acc_ref[0:1, :] + jnp.sum(d_oh, axis=0, keepdims=True)
    acc_ref[1:2, :] = acc_ref[1:2, :] + jnp.sum(a_oh, axis=0, keepdims=True)
    acc_ref[2:3, :] = acc_ref[2:3, :] + jnp.sum(probs, axis=0, keepdims=True)
    acc_ref[3:4, :] = acc_ref[3:4, :] + jnp.sum(lse * lse)

    @pl.when(i == nb - 1)
    def _():
        sf = jnp.float32(S)
        load = acc_ref[0:1, :] / sf
        imp = acc_ref[2:3, :] / sf
        bal = EXPERTS_WEIGHT * jnp.sum(load * imp)
        rl = ROUTER_WEIGHT * jnp.max(acc_ref[3:4, :]) / sf
        loss_ref[...] = jnp.reshape(rl + bal, (1, 1))


def _ffn_body(x_ref, w1_ref, b1_ref, w2_ref, b2_ref, y_ref):
    j = pl.program_id(1)
    x = x_ref[0]
    h1 = lax.dot_general(x, w1_ref[0], (((1,), (1,)), ((), ())),
                         preferred_element_type=jnp.float32)
    h1 = jax.nn.gelu(h1 + b1_ref[0], approximate=False)
    contrib = lax.dot_general(h1, w2_ref[0], (((1,), (1,)), ((), ())),
                              preferred_element_type=jnp.float32)

    @pl.when(j == 0)
    def _():
        y_ref[0] = contrib + b2_ref[0]

    @pl.when(j > 0)
    def _():
        y_ref[0] = y_ref[0] + contrib

    @pl.when(j == NJ - 1)
    def _():
        row = lax.broadcasted_iota(jnp.int32, (RPE, D), 0)
        y_ref[0] = jnp.where(row < CAP, y_ref[0], 0.0)


def _sc_wid():
    return lax.axis_index("s") * 2 + lax.axis_index("c")


def _dispatch_body(h_hbm, dst_hbm, x_hbm, idx_v, buf_v, sem):
    wid = _sc_wid()
    pltpu.sync_copy(dst_hbm.at[wid], idx_v)
    for c in range(NCH):
        base = wid * (NCH * CH) + c * CH
        pltpu.sync_copy(h_hbm.at[pl.ds(base, CH)], buf_v)
        pltpu.async_copy(buf_v, x_hbm.at[idx_v.at[c]], sem).wait()


def _combine_body(y_hbm, dst_hbm, o_hbm, idx_v, buf_v, sem):
    wid = _sc_wid()
    pltpu.sync_copy(dst_hbm.at[wid], idx_v)
    for c in range(NCH):
        base = wid * (NCH * CH) + c * CH
        pltpu.async_copy(y_hbm.at[idx_v.at[c]], buf_v, sem).wait()
        pltpu.sync_copy(buf_v, o_hbm.at[pl.ds(base, CH)])


def _make_router():
    return pl.pallas_call(
        _router_body,
        grid=(S // TB,),
        in_specs=[
            pl.BlockSpec((TB, D), lambda i: (i, 0)),
            pl.BlockSpec((D, LW), lambda i: (0, 0)),
        ],
        out_specs=[
            pl.BlockSpec((TB, 1), lambda i: (i, 0)),
            pl.BlockSpec((1, 1), lambda i: (0, 0)),
        ],
        out_shape=[
            jax.ShapeDtypeStruct((S, 1), jnp.int32),
            jax.ShapeDtypeStruct((1, 1), jnp.float32),
        ],
        scratch_shapes=[pltpu.VMEM((8, LW), jnp.float32)],
        compiler_params=pltpu.CompilerParams(
            dimension_semantics=("arbitrary",)),
    )


def _make_ffn():
    return pl.pallas_call(
        _ffn_body,
        grid=(E, NJ),
        in_specs=[
            pl.BlockSpec((1, RPE, D), lambda e, j: (e, 0, 0)),
            pl.BlockSpec((1, DFB, D), lambda e, j: (e, j, 0)),
            pl.BlockSpec((1, 1, DFB), lambda e, j: (e, 0, j)),
            pl.BlockSpec((1, D, DFB), lambda e, j: (e, 0, j)),
            pl.BlockSpec((1, 1, D), lambda e, j: (e, 0, 0)),
        ],
        out_specs=pl.BlockSpec((1, RPE, D), lambda e, j: (e, 0, 0)),
        out_shape=jax.ShapeDtypeStruct((E, RPE, D), jnp.float32),
        compiler_params=pltpu.CompilerParams(
            dimension_semantics=("arbitrary", "arbitrary")),
    )


_SC_MESH = plsc.VectorSubcoreMesh(core_axis_name="c", subcore_axis_name="s")


def _make_dispatch():
    return functools.partial(
        pl.kernel,
        out_type=jax.ShapeDtypeStruct((NROWS, D), jnp.float32),
        mesh=_SC_MESH,
        scratch_types=[
            pltpu.VMEM((NCH, CH), jnp.int32),
            pltpu.VMEM((CH, D), jnp.float32),
            pltpu.SemaphoreType.DMA,
        ],
    )(_dispatch_body)


def _make_combine():
    return functools.partial(
        pl.kernel,
        out_type=jax.ShapeDtypeStruct((S, D), jnp.float32),
        mesh=_SC_MESH,
        scratch_types=[
            pltpu.VMEM((NCH, CH), jnp.int32),
            pltpu.VMEM((CH, D), jnp.float32),
            pltpu.SemaphoreType.DMA,
        ],
    )(_combine_body)


def kernel(H, gate_W, fc1_w, fc1_b, fc2_w, fc2_b):
    b, t, d = H.shape
    h2 = H.reshape(S, D)
    gwt = jnp.zeros((D, LW), jnp.float32).at[:, :E].set(gate_W.T)

    dst, loss = _make_router()(h2, gwt)
    dst3 = dst.reshape(NW, NCH, CH)

    x = _make_dispatch()(h2, dst3)
    y = _make_ffn()(x.reshape(E, RPE, D), fc1_w,
                    fc1_b.reshape(E, 1, DFF), fc2_w, fc2_b.reshape(E, 1, D))
    out = _make_combine()(y.reshape(NROWS, D), dst3)
    return out.reshape(b, t, d), loss[0, 0]


# speedup vs baseline: 5.6021x; 5.6021x over previous
import functools

import jax
import jax.numpy as jnp
from jax import lax
from jax.experimental import pallas as pl
from jax.experimental.pallas import tpu as pltpu
from jax.experimental.pallas import tpu_sc as plsc

S = 4096
D = 1024
E = 8
DFF = 4096
CAP = 640
RPE = 656
NROWS = E * RPE
SENT = 648
TB = 512
LW = 128
NJ = 4
DFB = DFF // NJ

NW = 32
CH = 64
NCH = (S // NW) // CH

ROUTER_WEIGHT = 0.001
EXPERTS_WEIGHT = 0.01


def _router_body(h_ref, gw_ref, dst_ref, loss_ref, acc_ref):
    i = pl.program_id(0)
    nb = pl.num_programs(0)

    @pl.when(i == 0)
    def _():
        acc_ref[...] = jnp.zeros_like(acc_ref)

    h = h_ref[...]
    logits = jnp.dot(h, gw_ref[...], preferred_element_type=jnp.float32)
    lane = lax.broadcasted_iota(jnp.int32, (TB, LW), 1)
    neg_inf = jnp.float32(-jnp.inf)
    logits = jnp.where(lane < E, logits, neg_inf)

    m1 = jnp.max(logits, axis=1, keepdims=True)
    i1 = jnp.min(jnp.where(logits == m1, lane, LW), axis=1, keepdims=True)
    l2 = jnp.where(lane == i1, neg_inf, logits)
    m2 = jnp.max(l2, axis=1, keepdims=True)
    i2 = jnp.min(jnp.where(l2 == m2, lane, LW), axis=1, keepdims=True)

    ex = jnp.exp(logits - m1)
    se = jnp.sum(ex, axis=1, keepdims=True)
    probs = ex / se
    lse = m1 + jnp.log(se)

    oh1 = (lane == i1).astype(jnp.float32)
    oh2 = (lane == i2).astype(jnp.float32)
    d_oh = oh1 + oh2

    r_i = lax.broadcasted_iota(jnp.int32, (TB, TB), 0)
    c_i = lax.broadcasted_iota(jnp.int32, (TB, TB), 1)
    tri = (r_i >= c_i).astype(jnp.float32)

    pos = jnp.dot(tri, d_oh, preferred_element_type=jnp.float32) + acc_ref[0:1, :]
    pos1 = jnp.sum(pos * oh1, axis=1, keepdims=True)
    pos2 = jnp.sum(pos * oh2, axis=1, keepdims=True)
    cand1 = jnp.where(pos1 <= CAP, i1, -1)
    cand2 = jnp.where(pos2 <= CAP, i2, -1)
    assigned = jnp.maximum(cand1, cand2)

    a_oh = (lane == assigned).astype(jnp.float32)
    spos = jnp.dot(tri, a_oh, preferred_element_type=jnp.float32) + acc_ref[1:2, :]
    slot = jnp.sum(spos * a_oh, axis=1, keepdims=True).astype(jnp.int32) - 1
    dst_ref[...] = jnp.where(assigned >= 0, assigned * RPE + slot, SENT)

    acc_ref[0:1, :] = acc_ref[0:1, :] + jnp.sum(d_oh, axis=0, keepdims=True)
    acc_ref[1:2, :] = acc_ref[1:2, :] + jnp.sum(a_oh, axis=0, keepdims=True)
    acc_ref[2:3, :] = acc_ref[2:3, :] + jnp.sum(probs, axis=0, keepdims=True)
    acc_ref[3:4, :] = acc_ref[3:4, :] + jnp.sum(lse * lse)

    @pl.when(i == nb - 1)
    def _():
        sf = jnp.float32(S)
        load = acc_ref[0:1, :] / sf
        imp = acc_ref[2:3, :] / sf
        bal = EXPERTS_WEIGHT * jnp.sum(load * imp)
        rl = ROUTER_WEIGHT * jnp.max(acc_ref[3:4, :]) / sf
        loss_ref[...] = jnp.reshape(rl + bal, (1, 1))


def _ffn_body(x_ref, w1_ref, b1_ref, w2_ref, b2_ref, y_ref):
    j = pl.program_id(1)
    x = x_ref[0]
    h1 = lax.dot_general(x, w1_ref[0], (((1,), (1,)), ((), ())),
                         preferred_element_type=jnp.float32)
    h1 = h1 + b1_ref[0]
    h1 = 0.5 * h1 * (1.0 + lax.erf(h1 * jnp.float32(0.7071067811865476)))
    contrib = lax.dot_general(h1, w2_ref[0], (((1,), (1,)), ((), ())),
                              preferred_element_type=jnp.float32)

    @pl.when(j == 0)
    def _():
        y_ref[0] = contrib + b2_ref[0]

    @pl.when(j > 0)
    def _():
        y_ref[0] = y_ref[0] + contrib

    @pl.when(j == NJ - 1)
    def _():
        row = lax.broadcasted_iota(jnp.int32, (RPE, D), 0)
        y_ref[0] = jnp.where(row < CAP, y_ref[0], 0.0)


def _sc_wid():
    return lax.axis_index("s") * 2 + lax.axis_index("c")


def _dispatch_body(h_hbm, dst_hbm, x_hbm, idx_v, buf_v, sem):
    wid = _sc_wid()
    pltpu.sync_copy(dst_hbm.at[wid], idx_v)
    for c in range(NCH):
        base = wid * (NCH * CH) + c * CH
        pltpu.sync_copy(h_hbm.at[pl.ds(base, CH)], buf_v)
        pltpu.async_copy(buf_v, x_hbm.at[idx_v.at[c]], sem).wait()


def _combine_body(y_hbm, dst_hbm, o_hbm, idx_v, buf_v, sem):
    wid = _sc_wid()
    pltpu.sync_copy(dst_hbm.at[wid], idx_v)
    for c in range(NCH):
        base = wid * (NCH * CH) + c * CH
        pltpu.async_copy(y_hbm.at[idx_v.at[c]], buf_v, sem).wait()
        pltpu.sync_copy(buf_v, o_hbm.at[pl.ds(base, CH)])


def _make_router():
    return pl.pallas_call(
        _router_body,
        grid=(S // TB,),
        in_specs=[
            pl.BlockSpec((TB, D), lambda i: (i, 0)),
            pl.BlockSpec((D, LW), lambda i: (0, 0)),
        ],
        out_specs=[
            pl.BlockSpec((TB, 1), lambda i: (i, 0)),
            pl.BlockSpec((1, 1), lambda i: (0, 0)),
        ],
        out_shape=[
            jax.ShapeDtypeStruct((S, 1), jnp.int32),
            jax.ShapeDtypeStruct((1, 1), jnp.float32),
        ],
        scratch_shapes=[pltpu.VMEM((8, LW), jnp.float32)],
        compiler_params=pltpu.CompilerParams(
            dimension_semantics=("arbitrary",)),
    )


def _make_ffn():
    return pl.pallas_call(
        _ffn_body,
        grid=(E, NJ),
        in_specs=[
            pl.BlockSpec((1, RPE, D), lambda e, j: (e, 0, 0)),
            pl.BlockSpec((1, DFB, D), lambda e, j: (e, j, 0)),
            pl.BlockSpec((1, 1, DFB), lambda e, j: (e, 0, j)),
            pl.BlockSpec((1, D, DFB), lambda e, j: (e, 0, j)),
            pl.BlockSpec((1, 1, D), lambda e, j: (e, 0, 0)),
        ],
        out_specs=pl.BlockSpec((1, RPE, D), lambda e, j: (e, 0, 0)),
        out_shape=jax.ShapeDtypeStruct((E, RPE, D), jnp.float32),
        compiler_params=pltpu.CompilerParams(
            dimension_semantics=("arbitrary", "arbitrary")),
    )


def _sc_mesh():
    return plsc.VectorSubcoreMesh(core_axis_name="c", subcore_axis_name="s")


def _make_dispatch():
    return functools.partial(
        pl.kernel,
        out_type=jax.ShapeDtypeStruct((NROWS, D), jnp.float32),
        mesh=_sc_mesh(),
        scratch_types=[
            pltpu.VMEM((NCH, CH), jnp.int32),
            pltpu.VMEM((CH, D), jnp.float32),
            pltpu.SemaphoreType.DMA,
        ],
    )(_dispatch_body)


def _make_combine():
    return functools.partial(
        pl.kernel,
        out_type=jax.ShapeDtypeStruct((S, D), jnp.float32),
        mesh=_sc_mesh(),
        scratch_types=[
            pltpu.VMEM((NCH, CH), jnp.int32),
            pltpu.VMEM((CH, D), jnp.float32),
            pltpu.SemaphoreType.DMA,
        ],
    )(_combine_body)


def kernel(H, gate_W, fc1_w, fc1_b, fc2_w, fc2_b):
    b, t, d = H.shape
    h2 = H.reshape(S, D)
    gwt = jnp.zeros((D, LW), jnp.float32).at[:, :E].set(gate_W.T)

    dst, loss = _make_router()(h2, gwt)
    dst3 = dst.reshape(NW, NCH, CH)

    x = _make_dispatch()(h2, dst3)
    y = _make_ffn()(x.reshape(E, RPE, D), fc1_w,
                    fc1_b.reshape(E, 1, DFF), fc2_w, fc2_b.reshape(E, 1, D))
    out = _make_combine()(y.reshape(NROWS, D), dst3)
    return out.reshape(b, t, d), loss[0, 0]
